# baseline (speedup 1.0000x reference)
import functools

import jax
import jax.numpy as jnp
from jax.experimental import pallas as pl
from jax.experimental.pallas import tpu as pltpu
from jax.experimental.pallas import tpu_sc as plsc

T = 8192
D = 1024
FF = 2048
E = 16
BT = 256
NTB = T // BT
BT1 = 1024
NTB1 = T // BT1
BT5 = 512
NTB5 = T // BT5
BR = 256
NRB = 80
RCAP = NRB * BR
NSH = 4096
OUT = 256
PREC = jax.lax.Precision.HIGHEST
MM_PREC = jax.lax.Precision.DEFAULT
NW = 32
CH = 64


def _k1_body(x_ref, wp_ref, bp_ref, wr_ref, br_ref,
             h_ref, ti_ref, tw_ref, c0_ref, c1_ref):
    x = x_ref[...]
    h = jnp.dot(x, wp_ref[...], precision=MM_PREC) + bp_ref[...]
    h_ref[...] = h
    logits = jnp.dot(h, wr_ref[...], precision=MM_PREC) + br_ref[...]
    s = jax.nn.sigmoid(logits)
    lane = jax.lax.broadcasted_iota(jnp.int32, (BT1, E), 1)
    grp = lane // 4
    neg = jnp.float32(-1e30)
    big = jnp.int32(1000)
    gs_cols = []
    for g in range(4):
        m = grp == g
        sg = jnp.where(m, s, neg)
        m1 = jnp.max(sg, axis=1, keepdims=True)
        i1 = jnp.min(jnp.where(sg == m1, lane, big), axis=1, keepdims=True)
        sg2 = jnp.where(lane == i1, neg, sg)
        m2 = jnp.max(sg2, axis=1, keepdims=True)
        gs_cols.append(m1 + m2)
    gs = jnp.concatenate(gs_cols, axis=1)
    lane4 = jax.lax.broadcasted_iota(jnp.int32, (BT1, 4), 1)
    gm1 = jnp.max(gs, axis=1, keepdims=True)
    g1 = jnp.min(jnp.where(gs == gm1, lane4, big), axis=1, keepdims=True)
    gs2 = jnp.where(lane4 == g1, neg, gs)
    gm2 = jnp.max(gs2, axis=1, keepdims=True)
    g2 = jnp.min(jnp.where(gs2 == gm2, lane4, big), axis=1, keepdims=True)
    gmask = (grp == g1) | (grp == g2)
    ms = jnp.where(gmask, s, 0.0)
    v1 = jnp.max(ms, axis=1, keepdims=True)
    j1 = jnp.min(jnp.where(ms == v1, lane, big), axis=1, keepdims=True)
    ms2 = jnp.where(lane == j1, neg, ms)
    v2 = jnp.max(ms2, axis=1, keepdims=True)
    j2 = jnp.min(jnp.where(ms2 == v2, lane, big), axis=1, keepdims=True)
    den = v1 + v2 + 1e-20
    tw_ref[...] = jnp.concatenate([v1 / den, v2 / den], axis=1)
    ti_ref[...] = jnp.concatenate([j1, j2], axis=1)
    oh0 = (lane == j1).astype(jnp.float32)
    oh1 = (lane == j2).astype(jnp.float32)
    c0_ref[...] = jnp.sum(oh0, axis=0, keepdims=True)[None]
    c1_ref[...] = jnp.sum(oh1, axis=0, keepdims=True)[None]


def _k1(x, wp, bp2, wr, br2):
    return pl.pallas_call(
        _k1_body,
        grid=(NTB1,),
        in_specs=[
            pl.BlockSpec((BT1, D), lambda b: (b, 0)),
            pl.BlockSpec((D, D), lambda b: (0, 0)),
            pl.BlockSpec((1, D), lambda b: (0, 0)),
            pl.BlockSpec((D, E), lambda b: (0, 0)),
            pl.BlockSpec((1, E), lambda b: (0, 0)),
        ],
        out_specs=[
            pl.BlockSpec((BT1, D), lambda b: (b, 0)),
            pl.BlockSpec((BT1, 2), lambda b: (b, 0)),
            pl.BlockSpec((BT1, 2), lambda b: (b, 0)),
            pl.BlockSpec((1, 1, E), lambda b: (b, 0, 0)),
            pl.BlockSpec((1, 1, E), lambda b: (b, 0, 0)),
        ],
        out_shape=[
            jax.ShapeDtypeStruct((T, D), jnp.float32),
            jax.ShapeDtypeStruct((T, 2), jnp.int32),
            jax.ShapeDtypeStruct((T, 2), jnp.float32),
            jax.ShapeDtypeStruct((NTB1, 1, E), jnp.float32),
            jax.ShapeDtypeStruct((NTB1, 1, E), jnp.float32),
        ],
    )(x, wp, bp2, wr, br2)


def _k2_body(ti_ref, c0_ref, c1_ref, p0_ref, p1_ref, eb_ref):
    b = pl.program_id(0)
    c0 = c0_ref[:, 0, :]
    c1 = c1_ref[:, 0, :]
    tot0 = jnp.sum(c0, axis=0, keepdims=True)
    tot = tot0 + jnp.sum(c1, axis=0, keepdims=True)
    nb = jnp.floor((tot + 255.0) * (1.0 / 256.0))
    r16 = jax.lax.broadcasted_iota(jnp.int32, (E, E), 0)
    col16 = jax.lax.broadcasted_iota(jnp.int32, (E, E), 1)
    mincl = (r16 <= col16).astype(jnp.float32)
    cum_incl = jnp.dot(nb, mincl, precision=PREC)
    row_off = 256.0 * (cum_incl - nb)
    bm = (jax.lax.broadcasted_iota(jnp.int32, (NTB1, 1), 0) < b).astype(
        jnp.float32)
    ex0 = jnp.sum(c0 * bm, axis=0, keepdims=True)
    ex1 = tot0 + jnp.sum(c1 * bm, axis=0, keepdims=True)
    lane = jax.lax.broadcasted_iota(jnp.int32, (BT1, E), 1)
    rr = jax.lax.broadcasted_iota(jnp.int32, (BT1, BT1), 0)
    cc = jax.lax.broadcasted_iota(jnp.int32, (BT1, BT1), 1)
    tri = (cc < rr).astype(jnp.float32)
    for k, p_ref, ex in ((0, p0_ref, ex0), (1, p1_ref, ex1)):
        idx = ti_ref[:, k:k + 1]
        oh = (lane == idx).astype(jnp.float32)
        rank_full = jnp.dot(tri, oh, precision=PREC)
        rank = jnp.sum(oh * rank_full, axis=1, keepdims=True)
        base = jnp.sum(oh * (row_off + ex), axis=1, keepdims=True)
        p_ref[...] = (base + rank).astype(jnp.int32)
    r256 = jax.lax.broadcasted_iota(jnp.int32, (256, 1), 0).astype(jnp.float32)
    ge = (r256 >= cum_incl).astype(jnp.float32)
    ebv = jnp.minimum(jnp.sum(ge, axis=1, keepdims=True), 15.0)
    eb_ref[...] = ebv.astype(jnp.int32)


def _k2(ti, c0, c1):
    return pl.pallas_call(
        _k2_body,
        grid=(NTB1,),
        in_specs=[
            pl.BlockSpec((BT1, 2), lambda b: (b, 0)),
            pl.BlockSpec((NTB1, 1, E), lambda b: (0, 0, 0)),
            pl.BlockSpec((NTB1, 1, E), lambda b: (0, 0, 0)),
        ],
        out_specs=[
            pl.BlockSpec((BT1, 1), lambda b: (b, 0)),
            pl.BlockSpec((BT1, 1), lambda b: (b, 0)),
            pl.BlockSpec((256, 1), lambda b: (0, 0)),
        ],
        out_shape=[
            jax.ShapeDtypeStruct((T, 1), jnp.int32),
            jax.ShapeDtypeStruct((T, 1), jnp.int32),
            jax.ShapeDtypeStruct((256, 1), jnp.int32),
        ],
    )(ti, c0, c1)


def _s1(h, p0, p1):
    mesh = plsc.VectorSubcoreMesh(core_axis_name="c", subcore_axis_name="s")

    @functools.partial(
        pl.kernel,
        out_type=jax.ShapeDtypeStruct((RCAP, D), jnp.float32),
        mesh=mesh,
        scratch_types=[
            pltpu.VMEM((CH,), jnp.int32),
            pltpu.VMEM((CH, D), jnp.float32),
            pltpu.SemaphoreType.DMA,
        ],
    )
    def sc_scatter(h_hbm, p0_hbm, p1_hbm, hs_hbm, idx_v, rows_v, sem):
        wid = jax.lax.axis_index("s") * 2 + jax.lax.axis_index("c")
        base = wid * (T // NW)
        for p_hbm in (p0_hbm, p1_hbm):
            for c in range(T // NW // CH):
                off = base + c * CH
                pltpu.sync_copy(p_hbm.at[pl.ds(off, CH)], idx_v)
                pltpu.sync_copy(h_hbm.at[pl.ds(off, CH)], rows_v)
                pltpu.async_copy(rows_v, hs_hbm.at[idx_v], sem).wait()

    return sc_scatter(h, p0, p1)


def _k3_body(eb_sref, hs_ref, wg_ref, bg_ref, wu_ref, bu_ref, wd_ref, bd_ref,
             os_ref):
    hsb = hs_ref[...].astype(jnp.bfloat16)
    ag = jnp.dot(hsb, wg_ref[0], precision=MM_PREC,
                 preferred_element_type=jnp.float32) + bg_ref[0]
    au = jnp.dot(hsb, wu_ref[0], precision=MM_PREC,
                 preferred_element_type=jnp.float32) + bu_ref[0]
    act = (ag * jax.nn.sigmoid(ag) * au).astype(jnp.bfloat16)
    os_ref[...] = jnp.dot(act, wd_ref[0], precision=MM_PREC,
                          preferred_element_type=jnp.float32) + bd_ref[0]


def _k3(eb, hs, wg, bg3, wu, bu3, wd, bd3):
    gs = pltpu.PrefetchScalarGridSpec(
        num_scalar_prefetch=1,
        grid=(NRB,),
        in_specs=[
            pl.BlockSpec((BR, D), lambda r, eb: (r, 0)),
            pl.BlockSpec((1, D, FF), lambda r, eb: (eb[r], 0, 0)),
            pl.BlockSpec((1, 1, FF), lambda r, eb: (eb[r], 0, 0)),
            pl.BlockSpec((1, D, FF), lambda r, eb: (eb[r], 0, 0)),
            pl.BlockSpec((1, 1, FF), lambda r, eb: (eb[r], 0, 0)),
            pl.BlockSpec((1, FF, D), lambda r, eb: (eb[r], 0, 0)),
            pl.BlockSpec((1, 1, D), lambda r, eb: (eb[r], 0, 0)),
        ],
        out_specs=pl.BlockSpec((BR, D), lambda r, eb: (r, 0)),
    )
    return pl.pallas_call(
        _k3_body,
        grid_spec=gs,
        out_shape=jax.ShapeDtypeStruct((RCAP, D), jnp.float32),
    )(eb, hs, wg, bg3, wu, bu3, wd, bd3)


def _s2(os, p0, p1):
    mesh = plsc.VectorSubcoreMesh(core_axis_name="c", subcore_axis_name="s")

    @functools.partial(
        pl.kernel,
        out_type=jax.ShapeDtypeStruct((2, T, D), jnp.float32),
        mesh=mesh,
        scratch_types=[
            pltpu.VMEM((CH,), jnp.int32),
            pltpu.VMEM((CH, D), jnp.float32),
            pltpu.SemaphoreType.DMA,
        ],
    )
    def sc_gather(os_hbm, p0_hbm, p1_hbm, g2_hbm, idx_v, rows_v, sem):
        wid = jax.lax.axis_index("s") * 2 + jax.lax.axis_index("c")
        base = wid * (T // NW)
        for k, p_hbm in ((0, p0_hbm), (1, p1_hbm)):
            for c in range(T // NW // CH):
                off = base + c * CH
                pltpu.sync_copy(p_hbm.at[pl.ds(off, CH)], idx_v)
                pltpu.async_copy(os_hbm.at[idx_v], rows_v, sem).wait()
                pltpu.sync_copy(rows_v, g2_hbm.at[k, pl.ds(off, CH)])

    return sc_gather(os, p0, p1)


def _k4_body(h_ref, wsg_ref, bsg_ref, wsu_ref, bsu_ref, wsd_ref, bsd_ref,
             sh_ref):
    f0 = pl.program_id(0) == 0
    hb = h_ref[...]
    ag = jnp.dot(hb, wsg_ref[...], precision=MM_PREC) + bsg_ref[...]
    au = jnp.dot(hb, wsu_ref[...], precision=MM_PREC) + bsu_ref[...]
    act = ag * jax.nn.sigmoid(ag) * au
    res = jnp.dot(act, wsd_ref[...], precision=MM_PREC)
    sh_ref[...] = (res + jnp.where(f0, 1.0, 0.0) * bsd_ref[...])[None]


NSF = 4


def _k4_half(h, wsg, bsg2, wsu, bsu2, wsd, bsd2, half):
    fch = NSH // NSF
    return pl.pallas_call(
        _k4_body,
        grid=(NSF // 2, NTB),
        in_specs=[
            pl.BlockSpec((BT, D), lambda f, t: (t, 0)),
            pl.BlockSpec((D, fch), lambda f, t: (0, f + 2 * half)),
            pl.BlockSpec((1, fch), lambda f, t: (0, f + 2 * half)),
            pl.BlockSpec((D, fch), lambda f, t: (0, f + 2 * half)),
            pl.BlockSpec((1, fch), lambda f, t: (0, f + 2 * half)),
            pl.BlockSpec((fch, D), lambda f, t: (f + 2 * half, 0)),
            pl.BlockSpec((1, D), lambda f, t: (0, 0)),
        ],
        out_specs=pl.BlockSpec((1, BT, D), lambda f, t: (f, t, 0)),
        out_shape=jax.ShapeDtypeStruct((2, T, D), jnp.float32),
    )(h, wsg, bsg2, wsu, bsu2, wsd, bsd2)


def _k5_body(g2_ref, sha_ref, shb_ref, tw_ref, wo1_ref, bo1_ref, wo2_ref,
             bo2_ref, out_ref):
    g = g2_ref[...]
    sa = sha_ref[...]
    sb = shb_ref[...]
    w = tw_ref[...]
    y = (w[:, 0:1] * g[0] + w[:, 1:2] * g[1]
         + sa[0] + sa[1] + sb[0] + sb[1])
    a = jnp.dot(y, wo1_ref[...], precision=MM_PREC) + bo1_ref[...]
    a = a * jax.nn.sigmoid(a)
    out_ref[...] = jnp.dot(a, wo2_ref[...], precision=MM_PREC) + bo2_ref[...]


def _k5(g2, sha, shb, tw, wo1, bo12, wo2, bo22):
    return pl.pallas_call(
        _k5_body,
        grid=(NTB5,),
        in_specs=[
            pl.BlockSpec((2, BT5, D), lambda t: (0, t, 0)),
            pl.BlockSpec((2, BT5, D), lambda t: (0, t, 0)),
            pl.BlockSpec((2, BT5, D), lambda t: (0, t, 0)),
            pl.BlockSpec((BT5, 2), lambda t: (t, 0)),
            pl.BlockSpec((D, FF), lambda t: (0, 0)),
            pl.BlockSpec((1, FF), lambda t: (0, 0)),
            pl.BlockSpec((FF, OUT), lambda t: (0, 0)),
            pl.BlockSpec((1, OUT), lambda t: (0, 0)),
        ],
        out_specs=pl.BlockSpec((BT5, OUT), lambda t: (t, 0)),
        out_shape=jax.ShapeDtypeStruct((T, OUT), jnp.float32),
    )(g2, sha, shb, tw, wo1, bo12, wo2, bo22)


def kernel(x, Wp, bp, Wr, br, Wg, bg, Wu, bu, Wd, bd,
           Wsg, bsg, Wsu, bsu, Wsd, bsd, Wo1, bo1, Wo2, bo2):
    h, ti, tw, c0, c1 = _k1(x, Wp, bp.reshape(1, D), Wr, br.reshape(1, E))
    p0, p1, eb = _k2(ti, c0, c1)
    p0f = p0.reshape(T)
    p1f = p1.reshape(T)
    ebf = eb.reshape(256)
    hs = _s1(h, p0f, p1f)
    os = _k3(ebf, hs, Wg.astype(jnp.bfloat16), bg.reshape(E, 1, FF),
             Wu.astype(jnp.bfloat16), bu.reshape(E, 1, FF),
             Wd.astype(jnp.bfloat16), bd.reshape(E, 1, D))
    g2 = _s2(os, p0f, p1f)
    bsd2 = bsd.reshape(1, D)
    sha = _k4_half(h, Wsg, bsg.reshape(1, NSH), Wsu, bsu.reshape(1, NSH),
                   Wsd, bsd2, 0)
    shb = _k4_half(h, Wsg, bsg.reshape(1, NSH), Wsu, bsu.reshape(1, NSH),
                   Wsd, jnp.zeros_like(bsd2), 1)
    return _k5(g2, sha, shb, tw, Wo1, bo1.reshape(1, FF),
               Wo2, bo2.reshape(1, OUT))


# speedup vs baseline: 1.1232x; 1.1232x over previous
import functools

import jax
import jax.numpy as jnp
from jax.experimental import pallas as pl
from jax.experimental.pallas import tpu as pltpu
from jax.experimental.pallas import tpu_sc as plsc

T = 8192
D = 1024
FF = 2048
E = 16
BT = 256
NTB = T // BT
BT1 = 1024
NTB1 = T // BT1
BT5 = 512
NTB5 = T // BT5
BR = 256
NRB = 80
RCAP = NRB * BR
NSH = 4096
OUT = 256
PREC = jax.lax.Precision.HIGHEST
MM_PREC = jax.lax.Precision.DEFAULT
NW = 32
CH = 64


def _k1_body(x_ref, wp_ref, bp_ref, wr_ref, br_ref,
             h_ref, ti_ref, tw_ref, c0_ref, c1_ref):
    x = x_ref[...]
    h = jnp.dot(x, wp_ref[...], precision=MM_PREC) + bp_ref[...]
    h_ref[...] = h
    logits = jnp.dot(h, wr_ref[...], precision=MM_PREC) + br_ref[...]
    s = jax.nn.sigmoid(logits)
    lane = jax.lax.broadcasted_iota(jnp.int32, (BT1, E), 1)
    grp = lane // 4
    neg = jnp.float32(-1e30)
    big = jnp.int32(1000)
    gs_cols = []
    for g in range(4):
        m = grp == g
        sg = jnp.where(m, s, neg)
        m1 = jnp.max(sg, axis=1, keepdims=True)
        i1 = jnp.min(jnp.where(sg == m1, lane, big), axis=1, keepdims=True)
        sg2 = jnp.where(lane == i1, neg, sg)
        m2 = jnp.max(sg2, axis=1, keepdims=True)
        gs_cols.append(m1 + m2)
    gs = jnp.concatenate(gs_cols, axis=1)
    lane4 = jax.lax.broadcasted_iota(jnp.int32, (BT1, 4), 1)
    gm1 = jnp.max(gs, axis=1, keepdims=True)
    g1 = jnp.min(jnp.where(gs == gm1, lane4, big), axis=1, keepdims=True)
    gs2 = jnp.where(lane4 == g1, neg, gs)
    gm2 = jnp.max(gs2, axis=1, keepdims=True)
    g2 = jnp.min(jnp.where(gs2 == gm2, lane4, big), axis=1, keepdims=True)
    gmask = (grp == g1) | (grp == g2)
    ms = jnp.where(gmask, s, 0.0)
    v1 = jnp.max(ms, axis=1, keepdims=True)
    j1 = jnp.min(jnp.where(ms == v1, lane, big), axis=1, keepdims=True)
    ms2 = jnp.where(lane == j1, neg, ms)
    v2 = jnp.max(ms2, axis=1, keepdims=True)
    j2 = jnp.min(jnp.where(ms2 == v2, lane, big), axis=1, keepdims=True)
    den = v1 + v2 + 1e-20
    tw_ref[...] = jnp.concatenate([v1 / den, v2 / den], axis=1)
    ti_ref[...] = jnp.concatenate([j1, j2], axis=1)
    oh0 = (lane == j1).astype(jnp.float32)
    oh1 = (lane == j2).astype(jnp.float32)
    c0_ref[...] = jnp.sum(oh0, axis=0, keepdims=True)[None]
    c1_ref[...] = jnp.sum(oh1, axis=0, keepdims=True)[None]


def _k1(x, wp, bp2, wr, br2):
    return pl.pallas_call(
        _k1_body,
        grid=(NTB1,),
        in_specs=[
            pl.BlockSpec((BT1, D), lambda b: (b, 0)),
            pl.BlockSpec((D, D), lambda b: (0, 0)),
            pl.BlockSpec((1, D), lambda b: (0, 0)),
            pl.BlockSpec((D, E), lambda b: (0, 0)),
            pl.BlockSpec((1, E), lambda b: (0, 0)),
        ],
        out_specs=[
            pl.BlockSpec((BT1, D), lambda b: (b, 0)),
            pl.BlockSpec((BT1, 2), lambda b: (b, 0)),
            pl.BlockSpec((BT1, 2), lambda b: (b, 0)),
            pl.BlockSpec((1, 1, E), lambda b: (b, 0, 0)),
            pl.BlockSpec((1, 1, E), lambda b: (b, 0, 0)),
        ],
        out_shape=[
            jax.ShapeDtypeStruct((T, D), jnp.float32),
            jax.ShapeDtypeStruct((T, 2), jnp.int32),
            jax.ShapeDtypeStruct((T, 2), jnp.float32),
            jax.ShapeDtypeStruct((NTB1, 1, E), jnp.float32),
            jax.ShapeDtypeStruct((NTB1, 1, E), jnp.float32),
        ],
    )(x, wp, bp2, wr, br2)


def _k2_body(ti_ref, c0_ref, c1_ref, p0_ref, p1_ref, eb_ref):
    b = pl.program_id(0)
    c0 = c0_ref[:, 0, :]
    c1 = c1_ref[:, 0, :]
    tot0 = jnp.sum(c0, axis=0, keepdims=True)
    tot = tot0 + jnp.sum(c1, axis=0, keepdims=True)
    nb = jnp.floor((tot + 255.0) * (1.0 / 256.0))
    r16 = jax.lax.broadcasted_iota(jnp.int32, (E, E), 0)
    col16 = jax.lax.broadcasted_iota(jnp.int32, (E, E), 1)
    mincl = (r16 <= col16).astype(jnp.float32)
    cum_incl = jnp.dot(nb, mincl, precision=PREC)
    row_off = 256.0 * (cum_incl - nb)
    bm = (jax.lax.broadcasted_iota(jnp.int32, (NTB1, 1), 0) < b).astype(
        jnp.float32)
    ex0 = jnp.sum(c0 * bm, axis=0, keepdims=True)
    ex1 = tot0 + jnp.sum(c1 * bm, axis=0, keepdims=True)
    lane = jax.lax.broadcasted_iota(jnp.int32, (BT1, E), 1)
    rr = jax.lax.broadcasted_iota(jnp.int32, (BT1, BT1), 0)
    cc = jax.lax.broadcasted_iota(jnp.int32, (BT1, BT1), 1)
    tri = (cc < rr).astype(jnp.float32)
    for k, p_ref, ex in ((0, p0_ref, ex0), (1, p1_ref, ex1)):
        idx = ti_ref[:, k:k + 1]
        oh = (lane == idx).astype(jnp.float32)
        rank_full = jnp.dot(tri, oh, precision=PREC)
        rank = jnp.sum(oh * rank_full, axis=1, keepdims=True)
        base = jnp.sum(oh * (row_off + ex), axis=1, keepdims=True)
        p_ref[...] = (base + rank).astype(jnp.int32)
    r256 = jax.lax.broadcasted_iota(jnp.int32, (256, 1), 0).astype(jnp.float32)
    ge = (r256 >= cum_incl).astype(jnp.float32)
    ebv = jnp.minimum(jnp.sum(ge, axis=1, keepdims=True), 15.0)
    eb_ref[...] = ebv.astype(jnp.int32)


def _k2(ti, c0, c1):
    return pl.pallas_call(
        _k2_body,
        grid=(NTB1,),
        in_specs=[
            pl.BlockSpec((BT1, 2), lambda b: (b, 0)),
            pl.BlockSpec((NTB1, 1, E), lambda b: (0, 0, 0)),
            pl.BlockSpec((NTB1, 1, E), lambda b: (0, 0, 0)),
        ],
        out_specs=[
            pl.BlockSpec((BT1, 1), lambda b: (b, 0)),
            pl.BlockSpec((BT1, 1), lambda b: (b, 0)),
            pl.BlockSpec((256, 1), lambda b: (0, 0)),
        ],
        out_shape=[
            jax.ShapeDtypeStruct((T, 1), jnp.int32),
            jax.ShapeDtypeStruct((T, 1), jnp.int32),
            jax.ShapeDtypeStruct((256, 1), jnp.int32),
        ],
    )(ti, c0, c1)


def _s1(h, p0, p1):
    mesh = plsc.VectorSubcoreMesh(core_axis_name="c", subcore_axis_name="s")

    @functools.partial(
        pl.kernel,
        out_type=jax.ShapeDtypeStruct((RCAP, D), jnp.float32),
        mesh=mesh,
        scratch_types=[
            pltpu.VMEM((CH,), jnp.int32),
            pltpu.VMEM((CH, D), jnp.float32),
            pltpu.SemaphoreType.DMA,
        ],
    )
    def sc_scatter(h_hbm, p0_hbm, p1_hbm, hs_hbm, idx_v, rows_v, sem):
        wid = jax.lax.axis_index("s") * 2 + jax.lax.axis_index("c")
        base = wid * (T // NW)
        for p_hbm in (p0_hbm, p1_hbm):
            for c in range(T // NW // CH):
                off = base + c * CH
                pltpu.sync_copy(p_hbm.at[pl.ds(off, CH)], idx_v)
                pltpu.sync_copy(h_hbm.at[pl.ds(off, CH)], rows_v)
                pltpu.async_copy(rows_v, hs_hbm.at[idx_v], sem).wait()

    return sc_scatter(h, p0, p1)


def _k3_body(eb_sref, hs_ref, wg_ref, bg_ref, wu_ref, bu_ref, wd_ref, bd_ref,
             os_ref):
    hsb = hs_ref[...]
    ag = jnp.dot(hsb, wg_ref[0], precision=MM_PREC) + bg_ref[0]
    au = jnp.dot(hsb, wu_ref[0], precision=MM_PREC) + bu_ref[0]
    act = ag * jax.nn.sigmoid(ag) * au
    os_ref[...] = jnp.dot(act, wd_ref[0], precision=MM_PREC) + bd_ref[0]


def _k3(eb, hs, wg, bg3, wu, bu3, wd, bd3):
    gs = pltpu.PrefetchScalarGridSpec(
        num_scalar_prefetch=1,
        grid=(NRB,),
        in_specs=[
            pl.BlockSpec((BR, D), lambda r, eb: (r, 0)),
            pl.BlockSpec((1, D, FF), lambda r, eb: (eb[r], 0, 0)),
            pl.BlockSpec((1, 1, FF), lambda r, eb: (eb[r], 0, 0)),
            pl.BlockSpec((1, D, FF), lambda r, eb: (eb[r], 0, 0)),
            pl.BlockSpec((1, 1, FF), lambda r, eb: (eb[r], 0, 0)),
            pl.BlockSpec((1, FF, D), lambda r, eb: (eb[r], 0, 0)),
            pl.BlockSpec((1, 1, D), lambda r, eb: (eb[r], 0, 0)),
        ],
        out_specs=pl.BlockSpec((BR, D), lambda r, eb: (r, 0)),
    )
    return pl.pallas_call(
        _k3_body,
        grid_spec=gs,
        out_shape=jax.ShapeDtypeStruct((RCAP, D), jnp.float32),
    )(eb, hs, wg, bg3, wu, bu3, wd, bd3)


def _s2(os, p0, p1):
    mesh = plsc.VectorSubcoreMesh(core_axis_name="c", subcore_axis_name="s")

    @functools.partial(
        pl.kernel,
        out_type=jax.ShapeDtypeStruct((2, T, D), jnp.float32),
        mesh=mesh,
        scratch_types=[
            pltpu.VMEM((CH,), jnp.int32),
            pltpu.VMEM((CH, D), jnp.float32),
            pltpu.SemaphoreType.DMA,
        ],
    )
    def sc_gather(os_hbm, p0_hbm, p1_hbm, g2_hbm, idx_v, rows_v, sem):
        wid = jax.lax.axis_index("s") * 2 + jax.lax.axis_index("c")
        base = wid * (T // NW)
        for k, p_hbm in ((0, p0_hbm), (1, p1_hbm)):
            for c in range(T // NW // CH):
                off = base + c * CH
                pltpu.sync_copy(p_hbm.at[pl.ds(off, CH)], idx_v)
                pltpu.async_copy(os_hbm.at[idx_v], rows_v, sem).wait()
                pltpu.sync_copy(rows_v, g2_hbm.at[k, pl.ds(off, CH)])

    return sc_gather(os, p0, p1)


def _k4_body(h_ref, wsg_ref, bsg_ref, wsu_ref, bsu_ref, wsd_ref, bsd_ref,
             sh_ref):
    f0 = pl.program_id(0) == 0
    hb = h_ref[...]
    ag = jnp.dot(hb, wsg_ref[...], precision=MM_PREC) + bsg_ref[...]
    au = jnp.dot(hb, wsu_ref[...], precision=MM_PREC) + bsu_ref[...]
    act = ag * jax.nn.sigmoid(ag) * au
    res = jnp.dot(act, wsd_ref[...], precision=MM_PREC)
    sh_ref[...] = (res + jnp.where(f0, 1.0, 0.0) * bsd_ref[...])[None]


NSF = 4


def _k4_half(h, wsg, bsg2, wsu, bsu2, wsd, bsd2, half):
    fch = NSH // NSF
    return pl.pallas_call(
        _k4_body,
        grid=(NSF // 2, NTB),
        in_specs=[
            pl.BlockSpec((BT, D), lambda f, t: (t, 0)),
            pl.BlockSpec((D, fch), lambda f, t: (0, f + 2 * half)),
            pl.BlockSpec((1, fch), lambda f, t: (0, f + 2 * half)),
            pl.BlockSpec((D, fch), lambda f, t: (0, f + 2 * half)),
            pl.BlockSpec((1, fch), lambda f, t: (0, f + 2 * half)),
            pl.BlockSpec((fch, D), lambda f, t: (f + 2 * half, 0)),
            pl.BlockSpec((1, D), lambda f, t: (0, 0)),
        ],
        out_specs=pl.BlockSpec((1, BT, D), lambda f, t: (f, t, 0)),
        out_shape=jax.ShapeDtypeStruct((2, T, D), jnp.float32),
    )(h, wsg, bsg2, wsu, bsu2, wsd, bsd2)


def _k5_body(g2_ref, sha_ref, shb_ref, tw_ref, wo1_ref, bo1_ref, wo2_ref,
             bo2_ref, out_ref):
    g = g2_ref[...]
    sa = sha_ref[...]
    sb = shb_ref[...]
    w = tw_ref[...]
    y = (w[:, 0:1] * g[0] + w[:, 1:2] * g[1]
         + sa[0] + sa[1] + sb[0] + sb[1])
    a = jnp.dot(y, wo1_ref[...], precision=MM_PREC) + bo1_ref[...]
    a = a * jax.nn.sigmoid(a)
    out_ref[...] = jnp.dot(a, wo2_ref[...], precision=MM_PREC) + bo2_ref[...]


def _k5(g2, sha, shb, tw, wo1, bo12, wo2, bo22):
    return pl.pallas_call(
        _k5_body,
        grid=(NTB5,),
        in_specs=[
            pl.BlockSpec((2, BT5, D), lambda t: (0, t, 0)),
            pl.BlockSpec((2, BT5, D), lambda t: (0, t, 0)),
            pl.BlockSpec((2, BT5, D), lambda t: (0, t, 0)),
            pl.BlockSpec((BT5, 2), lambda t: (t, 0)),
            pl.BlockSpec((D, FF), lambda t: (0, 0)),
            pl.BlockSpec((1, FF), lambda t: (0, 0)),
            pl.BlockSpec((FF, OUT), lambda t: (0, 0)),
            pl.BlockSpec((1, OUT), lambda t: (0, 0)),
        ],
        out_specs=pl.BlockSpec((BT5, OUT), lambda t: (t, 0)),
        out_shape=jax.ShapeDtypeStruct((T, OUT), jnp.float32),
    )(g2, sha, shb, tw, wo1, bo12, wo2, bo22)


def kernel(x, Wp, bp, Wr, br, Wg, bg, Wu, bu, Wd, bd,
           Wsg, bsg, Wsu, bsu, Wsd, bsd, Wo1, bo1, Wo2, bo2):
    h, ti, tw, c0, c1 = _k1(x, Wp, bp.reshape(1, D), Wr, br.reshape(1, E))
    p0, p1, eb = _k2(ti, c0, c1)
    p0f = p0.reshape(T)
    p1f = p1.reshape(T)
    ebf = eb.reshape(256)
    hs = _s1(h, p0f, p1f)
    os = _k3(ebf, hs, Wg, bg.reshape(E, 1, FF), Wu, bu.reshape(E, 1, FF),
             Wd, bd.reshape(E, 1, D))
    g2 = _s2(os, p0f, p1f)
    bsd2 = bsd.reshape(1, D)
    sha = _k4_half(h, Wsg, bsg.reshape(1, NSH), Wsu, bsu.reshape(1, NSH),
                   Wsd, bsd2, 0)
    shb = _k4_half(h, Wsg, bsg.reshape(1, NSH), Wsu, bsu.reshape(1, NSH),
                   Wsd, jnp.zeros_like(bsd2), 1)
    return _k5(g2, sha, shb, tw, Wo1, bo1.reshape(1, FF),
               Wo2, bo2.reshape(1, OUT))


# speedup vs baseline: 1.1414x; 1.0162x over previous
import functools

import jax
import jax.numpy as jnp
from jax.experimental import pallas as pl
from jax.experimental.pallas import tpu as pltpu
from jax.experimental.pallas import tpu_sc as plsc

T = 8192
D = 1024
FF = 2048
E = 16
BT = 256
NTB = T // BT
BT1 = 1024
NTB1 = T // BT1
BT5 = 512
NTB5 = T // BT5
BR = 256
NRB = 80
RCAP = NRB * BR
NSH = 4096
OUT = 256
PREC = jax.lax.Precision.HIGHEST
MM_PREC = jax.lax.Precision.DEFAULT
NW = 32
CH = 64


def _k1_body(x_ref, wp_ref, bp_ref, wr_ref, br_ref,
             h_ref, ti_ref, tw_ref, c0_ref, c1_ref):
    x = x_ref[...]
    h = jnp.dot(x, wp_ref[...], precision=MM_PREC) + bp_ref[...]
    h_ref[...] = h
    logits = jnp.dot(h, wr_ref[...], precision=MM_PREC) + br_ref[...]
    s = jax.nn.sigmoid(logits)
    lane = jax.lax.broadcasted_iota(jnp.int32, (BT1, E), 1)
    grp = lane // 4
    neg = jnp.float32(-1e30)
    big = jnp.int32(1000)
    gs_cols = []
    for g in range(4):
        m = grp == g
        sg = jnp.where(m, s, neg)
        m1 = jnp.max(sg, axis=1, keepdims=True)
        i1 = jnp.min(jnp.where(sg == m1, lane, big), axis=1, keepdims=True)
        sg2 = jnp.where(lane == i1, neg, sg)
        m2 = jnp.max(sg2, axis=1, keepdims=True)
        gs_cols.append(m1 + m2)
    gs = jnp.concatenate(gs_cols, axis=1)
    lane4 = jax.lax.broadcasted_iota(jnp.int32, (BT1, 4), 1)
    gm1 = jnp.max(gs, axis=1, keepdims=True)
    g1 = jnp.min(jnp.where(gs == gm1, lane4, big), axis=1, keepdims=True)
    gs2 = jnp.where(lane4 == g1, neg, gs)
    gm2 = jnp.max(gs2, axis=1, keepdims=True)
    g2 = jnp.min(jnp.where(gs2 == gm2, lane4, big), axis=1, keepdims=True)
    gmask = (grp == g1) | (grp == g2)
    ms = jnp.where(gmask, s, 0.0)
    v1 = jnp.max(ms, axis=1, keepdims=True)
    j1 = jnp.min(jnp.where(ms == v1, lane, big), axis=1, keepdims=True)
    ms2 = jnp.where(lane == j1, neg, ms)
    v2 = jnp.max(ms2, axis=1, keepdims=True)
    j2 = jnp.min(jnp.where(ms2 == v2, lane, big), axis=1, keepdims=True)
    den = v1 + v2 + 1e-20
    tw_ref[...] = jnp.concatenate([v1 / den, v2 / den], axis=1)
    ti_ref[...] = jnp.concatenate([j1, j2], axis=1)
    oh0 = (lane == j1).astype(jnp.float32)
    oh1 = (lane == j2).astype(jnp.float32)
    c0_ref[...] = jnp.sum(oh0, axis=0, keepdims=True)[None]
    c1_ref[...] = jnp.sum(oh1, axis=0, keepdims=True)[None]


def _k1(x, wp, bp2, wr, br2):
    return pl.pallas_call(
        _k1_body,
        grid=(NTB1,),
        in_specs=[
            pl.BlockSpec((BT1, D), lambda b: (b, 0)),
            pl.BlockSpec((D, D), lambda b: (0, 0)),
            pl.BlockSpec((1, D), lambda b: (0, 0)),
            pl.BlockSpec((D, E), lambda b: (0, 0)),
            pl.BlockSpec((1, E), lambda b: (0, 0)),
        ],
        out_specs=[
            pl.BlockSpec((BT1, D), lambda b: (b, 0)),
            pl.BlockSpec((BT1, 2), lambda b: (b, 0)),
            pl.BlockSpec((BT1, 2), lambda b: (b, 0)),
            pl.BlockSpec((1, 1, E), lambda b: (b, 0, 0)),
            pl.BlockSpec((1, 1, E), lambda b: (b, 0, 0)),
        ],
        out_shape=[
            jax.ShapeDtypeStruct((T, D), jnp.float32),
            jax.ShapeDtypeStruct((T, 2), jnp.int32),
            jax.ShapeDtypeStruct((T, 2), jnp.float32),
            jax.ShapeDtypeStruct((NTB1, 1, E), jnp.float32),
            jax.ShapeDtypeStruct((NTB1, 1, E), jnp.float32),
        ],
    )(x, wp, bp2, wr, br2)


def _k2_body(ti_ref, c0_ref, c1_ref, p0_ref, p1_ref, eb_ref):
    b = pl.program_id(0)
    c0 = c0_ref[:, 0, :]
    c1 = c1_ref[:, 0, :]
    tot0 = jnp.sum(c0, axis=0, keepdims=True)
    tot = tot0 + jnp.sum(c1, axis=0, keepdims=True)
    nb = jnp.floor((tot + 255.0) * (1.0 / 256.0))
    r16 = jax.lax.broadcasted_iota(jnp.int32, (E, E), 0)
    col16 = jax.lax.broadcasted_iota(jnp.int32, (E, E), 1)
    mincl = (r16 <= col16).astype(jnp.float32)
    cum_incl = jnp.dot(nb, mincl, precision=PREC)
    row_off = 256.0 * (cum_incl - nb)
    bm = (jax.lax.broadcasted_iota(jnp.int32, (NTB1, 1), 0) < b).astype(
        jnp.float32)
    ex0 = jnp.sum(c0 * bm, axis=0, keepdims=True)
    ex1 = tot0 + jnp.sum(c1 * bm, axis=0, keepdims=True)
    lane = jax.lax.broadcasted_iota(jnp.int32, (BT1, E), 1)
    rr = jax.lax.broadcasted_iota(jnp.int32, (BT1, BT1), 0)
    cc = jax.lax.broadcasted_iota(jnp.int32, (BT1, BT1), 1)
    tri = (cc < rr).astype(jnp.float32)
    for k, p_ref, ex in ((0, p0_ref, ex0), (1, p1_ref, ex1)):
        idx = ti_ref[:, k:k + 1]
        oh = (lane == idx).astype(jnp.float32)
        rank_full = jnp.dot(tri, oh, precision=PREC)
        rank = jnp.sum(oh * rank_full, axis=1, keepdims=True)
        base = jnp.sum(oh * (row_off + ex), axis=1, keepdims=True)
        p_ref[...] = (base + rank).astype(jnp.int32)
    r256 = jax.lax.broadcasted_iota(jnp.int32, (256, 1), 0).astype(jnp.float32)
    ge = (r256 >= cum_incl).astype(jnp.float32)
    eb_ref[...] = jnp.sum(ge, axis=1, keepdims=True).astype(jnp.int32)


def _k2(ti, c0, c1):
    return pl.pallas_call(
        _k2_body,
        grid=(NTB1,),
        in_specs=[
            pl.BlockSpec((BT1, 2), lambda b: (b, 0)),
            pl.BlockSpec((NTB1, 1, E), lambda b: (0, 0, 0)),
            pl.BlockSpec((NTB1, 1, E), lambda b: (0, 0, 0)),
        ],
        out_specs=[
            pl.BlockSpec((BT1, 1), lambda b: (b, 0)),
            pl.BlockSpec((BT1, 1), lambda b: (b, 0)),
            pl.BlockSpec((256, 1), lambda b: (0, 0)),
        ],
        out_shape=[
            jax.ShapeDtypeStruct((T, 1), jnp.int32),
            jax.ShapeDtypeStruct((T, 1), jnp.int32),
            jax.ShapeDtypeStruct((256, 1), jnp.int32),
        ],
    )(ti, c0, c1)


def _s1(h, p0, p1):
    mesh = plsc.VectorSubcoreMesh(core_axis_name="c", subcore_axis_name="s")

    @functools.partial(
        pl.kernel,
        out_type=jax.ShapeDtypeStruct((RCAP, D), jnp.float32),
        mesh=mesh,
        scratch_types=[
            pltpu.VMEM((CH,), jnp.int32),
            pltpu.VMEM((CH, D), jnp.float32),
            pltpu.SemaphoreType.DMA,
        ],
    )
    def sc_scatter(h_hbm, p0_hbm, p1_hbm, hs_hbm, idx_v, rows_v, sem):
        wid = jax.lax.axis_index("s") * 2 + jax.lax.axis_index("c")
        base = wid * (T // NW)
        for p_hbm in (p0_hbm, p1_hbm):
            for c in range(T // NW // CH):
                off = base + c * CH
                pltpu.sync_copy(p_hbm.at[pl.ds(off, CH)], idx_v)
                pltpu.sync_copy(h_hbm.at[pl.ds(off, CH)], rows_v)
                pltpu.async_copy(rows_v, hs_hbm.at[idx_v], sem).wait()

    return sc_scatter(h, p0, p1)


def _k3_body(eb_sref, hs_ref, wg_ref, bg_ref, wu_ref, bu_ref, wd_ref, bd_ref,
             os_ref):
    @pl.when(eb_sref[pl.program_id(0)] < E)
    def _():
        hsb = hs_ref[...]
        ag = jnp.dot(hsb, wg_ref[0], precision=MM_PREC) + bg_ref[0]
        au = jnp.dot(hsb, wu_ref[0], precision=MM_PREC) + bu_ref[0]
        act = ag * jax.nn.sigmoid(ag) * au
        os_ref[...] = jnp.dot(act, wd_ref[0], precision=MM_PREC) + bd_ref[0]


def _k3(eb, hs, wg, bg3, wu, bu3, wd, bd3):
    gs = pltpu.PrefetchScalarGridSpec(
        num_scalar_prefetch=1,
        grid=(NRB,),
        in_specs=[
            pl.BlockSpec((BR, D), lambda r, eb: (r, 0)),
            pl.BlockSpec((1, D, FF),
                         lambda r, eb: (jnp.minimum(eb[r], E - 1), 0, 0)),
            pl.BlockSpec((1, 1, FF),
                         lambda r, eb: (jnp.minimum(eb[r], E - 1), 0, 0)),
            pl.BlockSpec((1, D, FF),
                         lambda r, eb: (jnp.minimum(eb[r], E - 1), 0, 0)),
            pl.BlockSpec((1, 1, FF),
                         lambda r, eb: (jnp.minimum(eb[r], E - 1), 0, 0)),
            pl.BlockSpec((1, FF, D),
                         lambda r, eb: (jnp.minimum(eb[r], E - 1), 0, 0)),
            pl.BlockSpec((1, 1, D),
                         lambda r, eb: (jnp.minimum(eb[r], E - 1), 0, 0)),
        ],
        out_specs=pl.BlockSpec((BR, D), lambda r, eb: (r, 0)),
    )
    return pl.pallas_call(
        _k3_body,
        grid_spec=gs,
        out_shape=jax.ShapeDtypeStruct((RCAP, D), jnp.float32),
    )(eb, hs, wg, bg3, wu, bu3, wd, bd3)


def _s2(os, p0, p1):
    mesh = plsc.VectorSubcoreMesh(core_axis_name="c", subcore_axis_name="s")

    @functools.partial(
        pl.kernel,
        out_type=jax.ShapeDtypeStruct((2, T, D), jnp.float32),
        mesh=mesh,
        scratch_types=[
            pltpu.VMEM((CH,), jnp.int32),
            pltpu.VMEM((CH, D), jnp.float32),
            pltpu.SemaphoreType.DMA,
        ],
    )
    def sc_gather(os_hbm, p0_hbm, p1_hbm, g2_hbm, idx_v, rows_v, sem):
        wid = jax.lax.axis_index("s") * 2 + jax.lax.axis_index("c")
        base = wid * (T // NW)
        for k, p_hbm in ((0, p0_hbm), (1, p1_hbm)):
            for c in range(T // NW // CH):
                off = base + c * CH
                pltpu.sync_copy(p_hbm.at[pl.ds(off, CH)], idx_v)
                pltpu.async_copy(os_hbm.at[idx_v], rows_v, sem).wait()
                pltpu.sync_copy(rows_v, g2_hbm.at[k, pl.ds(off, CH)])

    return sc_gather(os, p0, p1)


def _k4_body(h_ref, wsg_ref, bsg_ref, wsu_ref, bsu_ref, wsd_ref, bsd_ref,
             sh_ref):
    f0 = pl.program_id(0) == 0
    hb = h_ref[...]
    ag = jnp.dot(hb, wsg_ref[...], precision=MM_PREC) + bsg_ref[...]
    au = jnp.dot(hb, wsu_ref[...], precision=MM_PREC) + bsu_ref[...]
    act = ag * jax.nn.sigmoid(ag) * au
    res = jnp.dot(act, wsd_ref[...], precision=MM_PREC)
    sh_ref[...] = (res + jnp.where(f0, 1.0, 0.0) * bsd_ref[...])[None]


NSF = 4


def _k4_half(h, wsg, bsg2, wsu, bsu2, wsd, bsd2, half):
    fch = NSH // NSF
    return pl.pallas_call(
        _k4_body,
        grid=(NSF // 2, NTB),
        in_specs=[
            pl.BlockSpec((BT, D), lambda f, t: (t, 0)),
            pl.BlockSpec((D, fch), lambda f, t: (0, f + 2 * half)),
            pl.BlockSpec((1, fch), lambda f, t: (0, f + 2 * half)),
            pl.BlockSpec((D, fch), lambda f, t: (0, f + 2 * half)),
            pl.BlockSpec((1, fch), lambda f, t: (0, f + 2 * half)),
            pl.BlockSpec((fch, D), lambda f, t: (f + 2 * half, 0)),
            pl.BlockSpec((1, D), lambda f, t: (0, 0)),
        ],
        out_specs=pl.BlockSpec((1, BT, D), lambda f, t: (f, t, 0)),
        out_shape=jax.ShapeDtypeStruct((2, T, D), jnp.float32),
    )(h, wsg, bsg2, wsu, bsu2, wsd, bsd2)


def _k5_body(g2_ref, sha_ref, shb_ref, tw_ref, wo1_ref, bo1_ref, wo2_ref,
             bo2_ref, out_ref):
    g = g2_ref[...]
    sa = sha_ref[...]
    sb = shb_ref[...]
    w = tw_ref[...]
    y = (w[:, 0:1] * g[0] + w[:, 1:2] * g[1]
         + sa[0] + sa[1] + sb[0] + sb[1])
    a = jnp.dot(y, wo1_ref[...], precision=MM_PREC) + bo1_ref[...]
    a = a * jax.nn.sigmoid(a)
    out_ref[...] = jnp.dot(a, wo2_ref[...], precision=MM_PREC) + bo2_ref[...]


def _k5(g2, sha, shb, tw, wo1, bo12, wo2, bo22):
    return pl.pallas_call(
        _k5_body,
        grid=(NTB5,),
        in_specs=[
            pl.BlockSpec((2, BT5, D), lambda t: (0, t, 0)),
            pl.BlockSpec((2, BT5, D), lambda t: (0, t, 0)),
            pl.BlockSpec((2, BT5, D), lambda t: (0, t, 0)),
            pl.BlockSpec((BT5, 2), lambda t: (t, 0)),
            pl.BlockSpec((D, FF), lambda t: (0, 0)),
            pl.BlockSpec((1, FF), lambda t: (0, 0)),
            pl.BlockSpec((FF, OUT), lambda t: (0, 0)),
            pl.BlockSpec((1, OUT), lambda t: (0, 0)),
        ],
        out_specs=pl.BlockSpec((BT5, OUT), lambda t: (t, 0)),
        out_shape=jax.ShapeDtypeStruct((T, OUT), jnp.float32),
    )(g2, sha, shb, tw, wo1, bo12, wo2, bo22)


def kernel(x, Wp, bp, Wr, br, Wg, bg, Wu, bu, Wd, bd,
           Wsg, bsg, Wsu, bsu, Wsd, bsd, Wo1, bo1, Wo2, bo2):
    h, ti, tw, c0, c1 = _k1(x, Wp, bp.reshape(1, D), Wr, br.reshape(1, E))
    bsd2 = bsd.reshape(1, D)
    sha = _k4_half(h, Wsg, bsg.reshape(1, NSH), Wsu, bsu.reshape(1, NSH),
                   Wsd, bsd2, 0)
    p0, p1, eb = _k2(ti, c0, c1)
    p0f = p0.reshape(T)
    p1f = p1.reshape(T)
    ebf = eb.reshape(256)
    hs = _s1(h, p0f, p1f)
    os = _k3(ebf, hs, Wg, bg.reshape(E, 1, FF), Wu, bu.reshape(E, 1, FF),
             Wd, bd.reshape(E, 1, D))
    g2 = _s2(os, p0f, p1f)
    shb = _k4_half(h, Wsg, bsg.reshape(1, NSH), Wsu, bsu.reshape(1, NSH),
                   Wsd, jnp.zeros_like(bsd2), 1)
    return _k5(g2, sha, shb, tw, Wo1, bo1.reshape(1, FF),
               Wo2, bo2.reshape(1, OUT))


# speedup vs baseline: 1.1778x; 1.0319x over previous
import functools

import jax
import jax.numpy as jnp
from jax.experimental import pallas as pl
from jax.experimental.pallas import tpu as pltpu
from jax.experimental.pallas import tpu_sc as plsc

T = 8192
D = 1024
FF = 2048
E = 16
BT = 256
NTB = T // BT
BT1 = 1024
NTB1 = T // BT1
BT2 = 256
NTB2 = T // BT2
BT5 = 512
NTB5 = T // BT5
BR = 256
NRB = 80
RCAP = NRB * BR
NSH = 4096
OUT = 256
PREC = jax.lax.Precision.HIGHEST
MM_PREC = jax.lax.Precision.DEFAULT
NW = 32
CH = 64


def _k1_body(x_ref, wp_ref, bp_ref, wr_ref, br_ref,
             h_ref, ti_ref, tw_ref, c0_ref, c1_ref):
    x = x_ref[...]
    h = jnp.dot(x, wp_ref[...], precision=MM_PREC) + bp_ref[...]
    h_ref[...] = h
    logits = jnp.dot(h, wr_ref[...], precision=MM_PREC) + br_ref[...]
    s = jax.nn.sigmoid(logits)
    lane = jax.lax.broadcasted_iota(jnp.int32, (BT1, E), 1)
    grp = lane // 4
    neg = jnp.float32(-1e30)
    big = jnp.int32(1000)
    gs_cols = []
    for g in range(4):
        m = grp == g
        sg = jnp.where(m, s, neg)
        m1 = jnp.max(sg, axis=1, keepdims=True)
        i1 = jnp.min(jnp.where(sg == m1, lane, big), axis=1, keepdims=True)
        sg2 = jnp.where(lane == i1, neg, sg)
        m2 = jnp.max(sg2, axis=1, keepdims=True)
        gs_cols.append(m1 + m2)
    gs = jnp.concatenate(gs_cols, axis=1)
    lane4 = jax.lax.broadcasted_iota(jnp.int32, (BT1, 4), 1)
    gm1 = jnp.max(gs, axis=1, keepdims=True)
    g1 = jnp.min(jnp.where(gs == gm1, lane4, big), axis=1, keepdims=True)
    gs2 = jnp.where(lane4 == g1, neg, gs)
    gm2 = jnp.max(gs2, axis=1, keepdims=True)
    g2 = jnp.min(jnp.where(gs2 == gm2, lane4, big), axis=1, keepdims=True)
    gmask = (grp == g1) | (grp == g2)
    ms = jnp.where(gmask, s, 0.0)
    v1 = jnp.max(ms, axis=1, keepdims=True)
    j1 = jnp.min(jnp.where(ms == v1, lane, big), axis=1, keepdims=True)
    ms2 = jnp.where(lane == j1, neg, ms)
    v2 = jnp.max(ms2, axis=1, keepdims=True)
    j2 = jnp.min(jnp.where(ms2 == v2, lane, big), axis=1, keepdims=True)
    den = v1 + v2 + 1e-20
    tw_ref[...] = jnp.concatenate([v1 / den, v2 / den], axis=1)
    ti_ref[...] = jnp.concatenate([j1, j2], axis=1)
    oh0 = (lane == j1).astype(jnp.float32)
    oh1 = (lane == j2).astype(jnp.float32)
    sub = jax.lax.broadcasted_iota(jnp.int32, (BT1, 1), 0) // BT2
    for oh, c_ref in ((oh0, c0_ref), (oh1, c1_ref)):
        rows = [jnp.sum(oh * (sub == j), axis=0, keepdims=True)
                for j in range(BT1 // BT2)]
        c_ref[...] = jnp.concatenate(rows, axis=0)[None]


def _k1(x, wp, bp2, wr, br2):
    return pl.pallas_call(
        _k1_body,
        grid=(NTB1,),
        in_specs=[
            pl.BlockSpec((BT1, D), lambda b: (b, 0)),
            pl.BlockSpec((D, D), lambda b: (0, 0)),
            pl.BlockSpec((1, D), lambda b: (0, 0)),
            pl.BlockSpec((D, E), lambda b: (0, 0)),
            pl.BlockSpec((1, E), lambda b: (0, 0)),
        ],
        out_specs=[
            pl.BlockSpec((BT1, D), lambda b: (b, 0)),
            pl.BlockSpec((BT1, 2), lambda b: (b, 0)),
            pl.BlockSpec((BT1, 2), lambda b: (b, 0)),
            pl.BlockSpec((1, BT1 // BT2, E), lambda b: (b, 0, 0)),
            pl.BlockSpec((1, BT1 // BT2, E), lambda b: (b, 0, 0)),
        ],
        out_shape=[
            jax.ShapeDtypeStruct((T, D), jnp.float32),
            jax.ShapeDtypeStruct((T, 2), jnp.int32),
            jax.ShapeDtypeStruct((T, 2), jnp.float32),
            jax.ShapeDtypeStruct((NTB1, BT1 // BT2, E), jnp.float32),
            jax.ShapeDtypeStruct((NTB1, BT1 // BT2, E), jnp.float32),
        ],
    )(x, wp, bp2, wr, br2)


def _k2_body(ti_ref, c0_ref, c1_ref, p0_ref, p1_ref, eb_ref):
    b = pl.program_id(0)
    c0 = c0_ref[:, 0, :]
    c1 = c1_ref[:, 0, :]
    tot0 = jnp.sum(c0, axis=0, keepdims=True)
    tot = tot0 + jnp.sum(c1, axis=0, keepdims=True)
    nb = jnp.floor((tot + 255.0) * (1.0 / 256.0))
    r16 = jax.lax.broadcasted_iota(jnp.int32, (E, E), 0)
    col16 = jax.lax.broadcasted_iota(jnp.int32, (E, E), 1)
    mincl = (r16 <= col16).astype(jnp.float32)
    cum_incl = jnp.dot(nb, mincl, precision=PREC)
    row_off = 256.0 * (cum_incl - nb)
    bm = (jax.lax.broadcasted_iota(jnp.int32, (NTB2, 1), 0) < b).astype(
        jnp.float32)
    ex0 = jnp.sum(c0 * bm, axis=0, keepdims=True)
    ex1 = tot0 + jnp.sum(c1 * bm, axis=0, keepdims=True)
    lane = jax.lax.broadcasted_iota(jnp.int32, (BT2, E), 1)
    rr = jax.lax.broadcasted_iota(jnp.int32, (BT2, BT2), 0)
    cc = jax.lax.broadcasted_iota(jnp.int32, (BT2, BT2), 1)
    tri = (cc < rr).astype(jnp.float32)
    for k, p_ref, ex in ((0, p0_ref, ex0), (1, p1_ref, ex1)):
        idx = ti_ref[:, k:k + 1]
        oh = (lane == idx).astype(jnp.float32)
        rank_full = jnp.dot(tri, oh, precision=PREC)
        rank = jnp.sum(oh * rank_full, axis=1, keepdims=True)
        base = jnp.sum(oh * (row_off + ex), axis=1, keepdims=True)
        p_ref[...] = (base + rank).astype(jnp.int32)
    r256 = jax.lax.broadcasted_iota(jnp.int32, (256, 1), 0).astype(jnp.float32)
    ge = (r256 >= cum_incl).astype(jnp.float32)
    eb_ref[...] = jnp.sum(ge, axis=1, keepdims=True).astype(jnp.int32)


def _k2(ti, c0, c1):
    return pl.pallas_call(
        _k2_body,
        grid=(NTB2,),
        in_specs=[
            pl.BlockSpec((BT2, 2), lambda b: (b, 0)),
            pl.BlockSpec((NTB2, 1, E), lambda b: (0, 0, 0)),
            pl.BlockSpec((NTB2, 1, E), lambda b: (0, 0, 0)),
        ],
        out_specs=[
            pl.BlockSpec((BT2, 1), lambda b: (b, 0)),
            pl.BlockSpec((BT2, 1), lambda b: (b, 0)),
            pl.BlockSpec((256, 1), lambda b: (0, 0)),
        ],
        out_shape=[
            jax.ShapeDtypeStruct((T, 1), jnp.int32),
            jax.ShapeDtypeStruct((T, 1), jnp.int32),
            jax.ShapeDtypeStruct((256, 1), jnp.int32),
        ],
    )(ti, c0, c1)


def _s1(h, p0, p1):
    mesh = plsc.VectorSubcoreMesh(core_axis_name="c", subcore_axis_name="s")

    @functools.partial(
        pl.kernel,
        out_type=jax.ShapeDtypeStruct((RCAP, D), jnp.float32),
        mesh=mesh,
        scratch_types=[
            pltpu.VMEM((CH,), jnp.int32),
            pltpu.VMEM((CH, D), jnp.float32),
            pltpu.SemaphoreType.DMA,
        ],
    )
    def sc_scatter(h_hbm, p0_hbm, p1_hbm, hs_hbm, idx_v, rows_v, sem):
        wid = jax.lax.axis_index("s") * 2 + jax.lax.axis_index("c")
        base = wid * (T // NW)
        for p_hbm in (p0_hbm, p1_hbm):
            for c in range(T // NW // CH):
                off = base + c * CH
                pltpu.sync_copy(p_hbm.at[pl.ds(off, CH)], idx_v)
                pltpu.sync_copy(h_hbm.at[pl.ds(off, CH)], rows_v)
                pltpu.async_copy(rows_v, hs_hbm.at[idx_v], sem).wait()

    return sc_scatter(h, p0, p1)


def _k3_body(eb_sref, hs_ref, wg_ref, bg_ref, wu_ref, bu_ref, wd_ref, bd_ref,
             os_ref):
    @pl.when(eb_sref[pl.program_id(0)] < E)
    def _():
        hsb = hs_ref[...]
        ag = jnp.dot(hsb, wg_ref[0], precision=MM_PREC) + bg_ref[0]
        au = jnp.dot(hsb, wu_ref[0], precision=MM_PREC) + bu_ref[0]
        act = ag * jax.nn.sigmoid(ag) * au
        os_ref[...] = jnp.dot(act, wd_ref[0], precision=MM_PREC) + bd_ref[0]


def _k3(eb, hs, wg, bg3, wu, bu3, wd, bd3):
    gs = pltpu.PrefetchScalarGridSpec(
        num_scalar_prefetch=1,
        grid=(NRB,),
        in_specs=[
            pl.BlockSpec((BR, D), lambda r, eb: (r, 0)),
            pl.BlockSpec((1, D, FF),
                         lambda r, eb: (jnp.minimum(eb[r], E - 1), 0, 0)),
            pl.BlockSpec((1, 1, FF),
                         lambda r, eb: (jnp.minimum(eb[r], E - 1), 0, 0)),
            pl.BlockSpec((1, D, FF),
                         lambda r, eb: (jnp.minimum(eb[r], E - 1), 0, 0)),
            pl.BlockSpec((1, 1, FF),
                         lambda r, eb: (jnp.minimum(eb[r], E - 1), 0, 0)),
            pl.BlockSpec((1, FF, D),
                         lambda r, eb: (jnp.minimum(eb[r], E - 1), 0, 0)),
            pl.BlockSpec((1, 1, D),
                         lambda r, eb: (jnp.minimum(eb[r], E - 1), 0, 0)),
        ],
        out_specs=pl.BlockSpec((BR, D), lambda r, eb: (r, 0)),
    )
    return pl.pallas_call(
        _k3_body,
        grid_spec=gs,
        out_shape=jax.ShapeDtypeStruct((RCAP, D), jnp.float32),
    )(eb, hs, wg, bg3, wu, bu3, wd, bd3)


def _s2(os, p0, p1):
    mesh = plsc.VectorSubcoreMesh(core_axis_name="c", subcore_axis_name="s")

    @functools.partial(
        pl.kernel,
        out_type=jax.ShapeDtypeStruct((2, T, D), jnp.float32),
        mesh=mesh,
        scratch_types=[
            pltpu.VMEM((CH,), jnp.int32),
            pltpu.VMEM((CH, D), jnp.float32),
            pltpu.SemaphoreType.DMA,
        ],
    )
    def sc_gather(os_hbm, p0_hbm, p1_hbm, g2_hbm, idx_v, rows_v, sem):
        wid = jax.lax.axis_index("s") * 2 + jax.lax.axis_index("c")
        base = wid * (T // NW)
        for k, p_hbm in ((0, p0_hbm), (1, p1_hbm)):
            for c in range(T // NW // CH):
                off = base + c * CH
                pltpu.sync_copy(p_hbm.at[pl.ds(off, CH)], idx_v)
                pltpu.async_copy(os_hbm.at[idx_v], rows_v, sem).wait()
                pltpu.sync_copy(rows_v, g2_hbm.at[k, pl.ds(off, CH)])

    return sc_gather(os, p0, p1)


def _k4_body(h_ref, wsg_ref, bsg_ref, wsu_ref, bsu_ref, wsd_ref, bsd_ref,
             sh_ref):
    f0 = pl.program_id(0) == 0
    hb = h_ref[...]
    ag = jnp.dot(hb, wsg_ref[...], precision=MM_PREC) + bsg_ref[...]
    au = jnp.dot(hb, wsu_ref[...], precision=MM_PREC) + bsu_ref[...]
    act = ag * jax.nn.sigmoid(ag) * au
    res = jnp.dot(act, wsd_ref[...], precision=MM_PREC)
    sh_ref[...] = (res + jnp.where(f0, 1.0, 0.0) * bsd_ref[...])[None]


NSF = 4


def _k4_half(h, wsg, bsg2, wsu, bsu2, wsd, bsd2, half):
    fch = NSH // NSF
    return pl.pallas_call(
        _k4_body,
        grid=(NSF // 2, NTB),
        in_specs=[
            pl.BlockSpec((BT, D), lambda f, t: (t, 0)),
            pl.BlockSpec((D, fch), lambda f, t: (0, f + 2 * half)),
            pl.BlockSpec((1, fch), lambda f, t: (0, f + 2 * half)),
            pl.BlockSpec((D, fch), lambda f, t: (0, f + 2 * half)),
            pl.BlockSpec((1, fch), lambda f, t: (0, f + 2 * half)),
            pl.BlockSpec((fch, D), lambda f, t: (f + 2 * half, 0)),
            pl.BlockSpec((1, D), lambda f, t: (0, 0)),
        ],
        out_specs=pl.BlockSpec((1, BT, D), lambda f, t: (f, t, 0)),
        out_shape=jax.ShapeDtypeStruct((2, T, D), jnp.float32),
    )(h, wsg, bsg2, wsu, bsu2, wsd, bsd2)


def _k5_body(g2_ref, sha_ref, shb_ref, tw_ref, wo1_ref, bo1_ref, wo2_ref,
             bo2_ref, out_ref):
    g = g2_ref[...]
    sa = sha_ref[...]
    sb = shb_ref[...]
    w = tw_ref[...]
    y = (w[:, 0:1] * g[0] + w[:, 1:2] * g[1]
         + sa[0] + sa[1] + sb[0] + sb[1])
    a = jnp.dot(y, wo1_ref[...], precision=MM_PREC) + bo1_ref[...]
    a = a * jax.nn.sigmoid(a)
    out_ref[...] = jnp.dot(a, wo2_ref[...], precision=MM_PREC) + bo2_ref[...]


def _k5(g2, sha, shb, tw, wo1, bo12, wo2, bo22):
    return pl.pallas_call(
        _k5_body,
        grid=(NTB5,),
        in_specs=[
            pl.BlockSpec((2, BT5, D), lambda t: (0, t, 0)),
            pl.BlockSpec((2, BT5, D), lambda t: (0, t, 0)),
            pl.BlockSpec((2, BT5, D), lambda t: (0, t, 0)),
            pl.BlockSpec((BT5, 2), lambda t: (t, 0)),
            pl.BlockSpec((D, FF), lambda t: (0, 0)),
            pl.BlockSpec((1, FF), lambda t: (0, 0)),
            pl.BlockSpec((FF, OUT), lambda t: (0, 0)),
            pl.BlockSpec((1, OUT), lambda t: (0, 0)),
        ],
        out_specs=pl.BlockSpec((BT5, OUT), lambda t: (t, 0)),
        out_shape=jax.ShapeDtypeStruct((T, OUT), jnp.float32),
    )(g2, sha, shb, tw, wo1, bo12, wo2, bo22)


def kernel(x, Wp, bp, Wr, br, Wg, bg, Wu, bu, Wd, bd,
           Wsg, bsg, Wsu, bsu, Wsd, bsd, Wo1, bo1, Wo2, bo2):
    h, ti, tw, c0, c1 = _k1(x, Wp, bp.reshape(1, D), Wr, br.reshape(1, E))
    bsd2 = bsd.reshape(1, D)
    sha = _k4_half(h, Wsg, bsg.reshape(1, NSH), Wsu, bsu.reshape(1, NSH),
                   Wsd, bsd2, 0)
    p0, p1, eb = _k2(ti, c0.reshape(NTB2, 1, E), c1.reshape(NTB2, 1, E))
    p0f = p0.reshape(T)
    p1f = p1.reshape(T)
    ebf = eb.reshape(256)
    hs = _s1(h, p0f, p1f)
    os = _k3(ebf, hs, Wg, bg.reshape(E, 1, FF), Wu, bu.reshape(E, 1, FF),
             Wd, bd.reshape(E, 1, D))
    g2 = _s2(os, p0f, p1f)
    shb = _k4_half(h, Wsg, bsg.reshape(1, NSH), Wsu, bsu.reshape(1, NSH),
                   Wsd, jnp.zeros_like(bsd2), 1)
    return _k5(g2, sha, shb, tw, Wo1, bo1.reshape(1, FF),
               Wo2, bo2.reshape(1, OUT))


# speedup vs baseline: 1.2101x; 1.0274x over previous
import functools

import jax
import jax.numpy as jnp
from jax.experimental import pallas as pl
from jax.experimental.pallas import tpu as pltpu
from jax.experimental.pallas import tpu_sc as plsc

T = 8192
D = 1024
FF = 2048
E = 16
BT = 256
NTB = T // BT
BT1 = 1024
NTB1 = T // BT1
BT2 = 256
NTB2 = T // BT2
BT5 = 512
NTB5 = T // BT5
BR = 256
NRB = 80
RCAP = NRB * BR
NSH = 4096
OUT = 256
PREC = jax.lax.Precision.HIGHEST
MM_PREC = jax.lax.Precision.DEFAULT
NW = 32
CH = 64


def _k1_body(x_ref, wp_ref, bp_ref, wr_ref, br_ref,
             h_ref, ti_ref, tw_ref, c0_ref, c1_ref):
    x = x_ref[...]
    h = jnp.dot(x, wp_ref[...], precision=MM_PREC) + bp_ref[...]
    h_ref[...] = h
    logits = jnp.dot(h, wr_ref[...], precision=MM_PREC) + br_ref[...]
    s = jax.nn.sigmoid(logits)
    lane = jax.lax.broadcasted_iota(jnp.int32, (BT1, E), 1)
    grp = lane // 4
    neg = jnp.float32(-1e30)
    big = jnp.int32(1000)
    gs_cols = []
    for g in range(4):
        m = grp == g
        sg = jnp.where(m, s, neg)
        m1 = jnp.max(sg, axis=1, keepdims=True)
        i1 = jnp.min(jnp.where(sg == m1, lane, big), axis=1, keepdims=True)
        sg2 = jnp.where(lane == i1, neg, sg)
        m2 = jnp.max(sg2, axis=1, keepdims=True)
        gs_cols.append(m1 + m2)
    gs = jnp.concatenate(gs_cols, axis=1)
    lane4 = jax.lax.broadcasted_iota(jnp.int32, (BT1, 4), 1)
    gm1 = jnp.max(gs, axis=1, keepdims=True)
    g1 = jnp.min(jnp.where(gs == gm1, lane4, big), axis=1, keepdims=True)
    gs2 = jnp.where(lane4 == g1, neg, gs)
    gm2 = jnp.max(gs2, axis=1, keepdims=True)
    g2 = jnp.min(jnp.where(gs2 == gm2, lane4, big), axis=1, keepdims=True)
    gmask = (grp == g1) | (grp == g2)
    ms = jnp.where(gmask, s, 0.0)
    v1 = jnp.max(ms, axis=1, keepdims=True)
    j1 = jnp.min(jnp.where(ms == v1, lane, big), axis=1, keepdims=True)
    ms2 = jnp.where(lane == j1, neg, ms)
    v2 = jnp.max(ms2, axis=1, keepdims=True)
    j2 = jnp.min(jnp.where(ms2 == v2, lane, big), axis=1, keepdims=True)
    den = v1 + v2 + 1e-20
    tw_ref[...] = jnp.concatenate([v1 / den, v2 / den], axis=1)
    ti_ref[...] = jnp.concatenate([j1, j2], axis=1)
    oh0 = (lane == j1).astype(jnp.float32)
    oh1 = (lane == j2).astype(jnp.float32)
    sub = jax.lax.broadcasted_iota(jnp.int32, (BT1, 1), 0) // BT2
    for oh, c_ref in ((oh0, c0_ref), (oh1, c1_ref)):
        rows = [jnp.sum(oh * (sub == j), axis=0, keepdims=True)
                for j in range(BT1 // BT2)]
        c_ref[...] = jnp.concatenate(rows, axis=0)[None]


def _k1(x, wp, bp2, wr, br2):
    return pl.pallas_call(
        _k1_body,
        grid=(NTB1,),
        in_specs=[
            pl.BlockSpec((BT1, D), lambda b: (b, 0)),
            pl.BlockSpec((D, D), lambda b: (0, 0)),
            pl.BlockSpec((1, D), lambda b: (0, 0)),
            pl.BlockSpec((D, E), lambda b: (0, 0)),
            pl.BlockSpec((1, E), lambda b: (0, 0)),
        ],
        out_specs=[
            pl.BlockSpec((BT1, D), lambda b: (b, 0)),
            pl.BlockSpec((BT1, 2), lambda b: (b, 0)),
            pl.BlockSpec((BT1, 2), lambda b: (b, 0)),
            pl.BlockSpec((1, BT1 // BT2, E), lambda b: (b, 0, 0)),
            pl.BlockSpec((1, BT1 // BT2, E), lambda b: (b, 0, 0)),
        ],
        out_shape=[
            jax.ShapeDtypeStruct((T, D), jnp.float32),
            jax.ShapeDtypeStruct((T, 2), jnp.int32),
            jax.ShapeDtypeStruct((T, 2), jnp.float32),
            jax.ShapeDtypeStruct((NTB1, BT1 // BT2, E), jnp.float32),
            jax.ShapeDtypeStruct((NTB1, BT1 // BT2, E), jnp.float32),
        ],
    )(x, wp, bp2, wr, br2)


def _k2_body(ti_ref, c0_ref, c1_ref, p0_ref, p1_ref, eb_ref):
    b = pl.program_id(0)
    c0 = c0_ref[:, 0, :]
    c1 = c1_ref[:, 0, :]
    tot0 = jnp.sum(c0, axis=0, keepdims=True)
    tot = tot0 + jnp.sum(c1, axis=0, keepdims=True)
    nb = jnp.floor((tot + 255.0) * (1.0 / 256.0))
    r16 = jax.lax.broadcasted_iota(jnp.int32, (E, E), 0)
    col16 = jax.lax.broadcasted_iota(jnp.int32, (E, E), 1)
    mincl = (r16 <= col16).astype(jnp.float32)
    cum_incl = jnp.dot(nb, mincl, precision=PREC)
    row_off = 256.0 * (cum_incl - nb)
    bm = (jax.lax.broadcasted_iota(jnp.int32, (NTB2, 1), 0) < b).astype(
        jnp.float32)
    ex0 = jnp.sum(c0 * bm, axis=0, keepdims=True)
    ex1 = tot0 + jnp.sum(c1 * bm, axis=0, keepdims=True)
    lane = jax.lax.broadcasted_iota(jnp.int32, (BT2, E), 1)
    rr = jax.lax.broadcasted_iota(jnp.int32, (BT2, BT2), 0)
    cc = jax.lax.broadcasted_iota(jnp.int32, (BT2, BT2), 1)
    tri = (cc < rr).astype(jnp.float32)
    for k, p_ref, ex in ((0, p0_ref, ex0), (1, p1_ref, ex1)):
        idx = ti_ref[:, k:k + 1]
        oh = (lane == idx).astype(jnp.float32)
        rank_full = jnp.dot(tri, oh, precision=PREC)
        rank = jnp.sum(oh * rank_full, axis=1, keepdims=True)
        base = jnp.sum(oh * (row_off + ex), axis=1, keepdims=True)
        p_ref[...] = (base + rank).astype(jnp.int32)
    r256 = jax.lax.broadcasted_iota(jnp.int32, (256, 1), 0).astype(jnp.float32)
    ge = (r256 >= cum_incl).astype(jnp.float32)
    eb_ref[...] = jnp.sum(ge, axis=1, keepdims=True).astype(jnp.int32)


def _k2(ti, c0, c1):
    return pl.pallas_call(
        _k2_body,
        grid=(NTB2,),
        in_specs=[
            pl.BlockSpec((BT2, 2), lambda b: (b, 0)),
            pl.BlockSpec((NTB2, 1, E), lambda b: (0, 0, 0)),
            pl.BlockSpec((NTB2, 1, E), lambda b: (0, 0, 0)),
        ],
        out_specs=[
            pl.BlockSpec((BT2, 1), lambda b: (b, 0)),
            pl.BlockSpec((BT2, 1), lambda b: (b, 0)),
            pl.BlockSpec((256, 1), lambda b: (0, 0)),
        ],
        out_shape=[
            jax.ShapeDtypeStruct((T, 1), jnp.int32),
            jax.ShapeDtypeStruct((T, 1), jnp.int32),
            jax.ShapeDtypeStruct((256, 1), jnp.int32),
        ],
    )(ti, c0, c1)


def _s1(h, p0, p1):
    mesh = plsc.VectorSubcoreMesh(core_axis_name="c", subcore_axis_name="s")

    @functools.partial(
        pl.kernel,
        out_type=jax.ShapeDtypeStruct((RCAP, D), jnp.float32),
        mesh=mesh,
        scratch_types=[
            pltpu.VMEM((CH,), jnp.int32),
            pltpu.VMEM((CH, D), jnp.float32),
            pltpu.SemaphoreType.DMA,
        ],
    )
    def sc_scatter(h_hbm, p0_hbm, p1_hbm, hs_hbm, idx_v, rows_v, sem):
        wid = jax.lax.axis_index("s") * 2 + jax.lax.axis_index("c")
        base = wid * (T // NW)
        for p_hbm in (p0_hbm, p1_hbm):
            for c in range(T // NW // CH):
                off = base + c * CH
                pltpu.sync_copy(p_hbm.at[pl.ds(off, CH)], idx_v)
                pltpu.sync_copy(h_hbm.at[pl.ds(off, CH)], rows_v)
                pltpu.async_copy(rows_v, hs_hbm.at[idx_v], sem).wait()

    return sc_scatter(h, p0, p1)


def _k3_body(eb_sref, hs_ref, wg_ref, bg_ref, wu_ref, bu_ref, wd_ref, bd_ref,
             os_ref):
    @pl.when(eb_sref[pl.program_id(0)] < E)
    def _():
        hsb = hs_ref[...]
        ag = jnp.dot(hsb, wg_ref[0], precision=MM_PREC) + bg_ref[0]
        au = jnp.dot(hsb, wu_ref[0], precision=MM_PREC) + bu_ref[0]
        act = ag * jax.nn.sigmoid(ag) * au
        os_ref[...] = jnp.dot(act, wd_ref[0], precision=MM_PREC) + bd_ref[0]


def _k3(eb, hs, wg, bg3, wu, bu3, wd, bd3):
    gs = pltpu.PrefetchScalarGridSpec(
        num_scalar_prefetch=1,
        grid=(NRB,),
        in_specs=[
            pl.BlockSpec((BR, D), lambda r, eb: (r, 0)),
            pl.BlockSpec((1, D, FF),
                         lambda r, eb: (jnp.minimum(eb[r], E - 1), 0, 0)),
            pl.BlockSpec((1, 1, FF),
                         lambda r, eb: (jnp.minimum(eb[r], E - 1), 0, 0)),
            pl.BlockSpec((1, D, FF),
                         lambda r, eb: (jnp.minimum(eb[r], E - 1), 0, 0)),
            pl.BlockSpec((1, 1, FF),
                         lambda r, eb: (jnp.minimum(eb[r], E - 1), 0, 0)),
            pl.BlockSpec((1, FF, D),
                         lambda r, eb: (jnp.minimum(eb[r], E - 1), 0, 0)),
            pl.BlockSpec((1, 1, D),
                         lambda r, eb: (jnp.minimum(eb[r], E - 1), 0, 0)),
        ],
        out_specs=pl.BlockSpec((BR, D), lambda r, eb: (r, 0)),
    )
    return pl.pallas_call(
        _k3_body,
        grid_spec=gs,
        out_shape=jax.ShapeDtypeStruct((RCAP, D), jnp.float32),
    )(eb, hs, wg, bg3, wu, bu3, wd, bd3)


def _s2(os, p0, p1):
    mesh = plsc.VectorSubcoreMesh(core_axis_name="c", subcore_axis_name="s")

    @functools.partial(
        pl.kernel,
        out_type=jax.ShapeDtypeStruct((2, T, D), jnp.float32),
        mesh=mesh,
        scratch_types=[
            pltpu.VMEM((CH,), jnp.int32),
            pltpu.VMEM((CH, D), jnp.float32),
            pltpu.SemaphoreType.DMA,
        ],
    )
    def sc_gather(os_hbm, p0_hbm, p1_hbm, g2_hbm, idx_v, rows_v, sem):
        wid = jax.lax.axis_index("s") * 2 + jax.lax.axis_index("c")
        base = wid * (T // NW)
        for k, p_hbm in ((0, p0_hbm), (1, p1_hbm)):
            for c in range(T // NW // CH):
                off = base + c * CH
                pltpu.sync_copy(p_hbm.at[pl.ds(off, CH)], idx_v)
                pltpu.async_copy(os_hbm.at[idx_v], rows_v, sem).wait()
                pltpu.sync_copy(rows_v, g2_hbm.at[k, pl.ds(off, CH)])

    return sc_gather(os, p0, p1)


def _k4_body(h_ref, wsg_ref, bsg_ref, wsu_ref, bsu_ref, wsd_ref, bsd_ref,
             sh_ref):
    f0 = pl.program_id(0) == 0
    hb = h_ref[...]
    ag = jnp.dot(hb, wsg_ref[...], precision=MM_PREC) + bsg_ref[...]
    au = jnp.dot(hb, wsu_ref[...], precision=MM_PREC) + bsu_ref[...]
    act = ag * jax.nn.sigmoid(ag) * au
    res = jnp.dot(act, wsd_ref[...], precision=MM_PREC)
    sh_ref[...] = (res + jnp.where(f0, 1.0, 0.0) * bsd_ref[...])[None]


NSF = 4


BT4 = 512
NTB4 = T // BT4


def _k4_half(h, wsg, bsg2, wsu, bsu2, wsd, bsd2, half):
    fch = NSH // NSF
    return pl.pallas_call(
        _k4_body,
        grid=(NSF // 2, NTB4),
        in_specs=[
            pl.BlockSpec((BT4, D), lambda f, t: (t, 0)),
            pl.BlockSpec((D, fch), lambda f, t: (0, f + 2 * half)),
            pl.BlockSpec((1, fch), lambda f, t: (0, f + 2 * half)),
            pl.BlockSpec((D, fch), lambda f, t: (0, f + 2 * half)),
            pl.BlockSpec((1, fch), lambda f, t: (0, f + 2 * half)),
            pl.BlockSpec((fch, D), lambda f, t: (f + 2 * half, 0)),
            pl.BlockSpec((1, D), lambda f, t: (0, 0)),
        ],
        out_specs=pl.BlockSpec((1, BT4, D), lambda f, t: (f, t, 0)),
        out_shape=jax.ShapeDtypeStruct((2, T, D), jnp.float32),
    )(h, wsg, bsg2, wsu, bsu2, wsd, bsd2)


def _k5_body(g2_ref, sha_ref, shb_ref, tw_ref, wo1_ref, bo1_ref, wo2_ref,
             bo2_ref, out_ref):
    g = g2_ref[...]
    sa = sha_ref[...]
    sb = shb_ref[...]
    w = tw_ref[...]
    y = (w[:, 0:1] * g[0] + w[:, 1:2] * g[1]
         + sa[0] + sa[1] + sb[0] + sb[1])
    a = jnp.dot(y, wo1_ref[...], precision=MM_PREC) + bo1_ref[...]
    a = a * jax.nn.sigmoid(a)
    out_ref[...] = jnp.dot(a, wo2_ref[...], precision=MM_PREC) + bo2_ref[...]


def _k5(g2, sha, shb, tw, wo1, bo12, wo2, bo22):
    return pl.pallas_call(
        _k5_body,
        grid=(NTB5,),
        in_specs=[
            pl.BlockSpec((2, BT5, D), lambda t: (0, t, 0)),
            pl.BlockSpec((2, BT5, D), lambda t: (0, t, 0)),
            pl.BlockSpec((2, BT5, D), lambda t: (0, t, 0)),
            pl.BlockSpec((BT5, 2), lambda t: (t, 0)),
            pl.BlockSpec((D, FF), lambda t: (0, 0)),
            pl.BlockSpec((1, FF), lambda t: (0, 0)),
            pl.BlockSpec((FF, OUT), lambda t: (0, 0)),
            pl.BlockSpec((1, OUT), lambda t: (0, 0)),
        ],
        out_specs=pl.BlockSpec((BT5, OUT), lambda t: (t, 0)),
        out_shape=jax.ShapeDtypeStruct((T, OUT), jnp.float32),
    )(g2, sha, shb, tw, wo1, bo12, wo2, bo22)


def kernel(x, Wp, bp, Wr, br, Wg, bg, Wu, bu, Wd, bd,
           Wsg, bsg, Wsu, bsu, Wsd, bsd, Wo1, bo1, Wo2, bo2):
    h, ti, tw, c0, c1 = _k1(x, Wp, bp.reshape(1, D), Wr, br.reshape(1, E))
    bsd2 = bsd.reshape(1, D)
    sha = _k4_half(h, Wsg, bsg.reshape(1, NSH), Wsu, bsu.reshape(1, NSH),
                   Wsd, bsd2, 0)
    p0, p1, eb = _k2(ti, c0.reshape(NTB2, 1, E), c1.reshape(NTB2, 1, E))
    p0f = p0.reshape(T)
    p1f = p1.reshape(T)
    ebf = eb.reshape(256)
    hs = _s1(h, p0f, p1f)
    os = _k3(ebf, hs, Wg, bg.reshape(E, 1, FF), Wu, bu.reshape(E, 1, FF),
             Wd, bd.reshape(E, 1, D))
    g2 = _s2(os, p0f, p1f)
    shb = _k4_half(h, Wsg, bsg.reshape(1, NSH), Wsu, bsu.reshape(1, NSH),
                   Wsd, jnp.zeros_like(bsd2), 1)
    return _k5(g2, sha, shb, tw, Wo1, bo1.reshape(1, FF),
               Wo2, bo2.reshape(1, OUT))
